# baseline (speedup 1.0000x reference)
import functools

import jax
import jax.numpy as jnp
from jax import lax
from jax.experimental import pallas as pl
from jax.experimental.pallas import tpu as pltpu
from jax.experimental.pallas import tpu_sc as plsc

ATOM_FDIM = 39
N_ATOMS = 10000
N_BONDS = 160000
MAX_NB = 6
HIDDEN = 256
OUT = 256
DEPTH = 3

_AF_PAD = 128
_AT_ROWS = N_ATOMS + 8

_NW = 32
_C = 40


def _sc_info():
    info = plsc.get_sparse_core_info()
    return info.num_cores, info.num_subcores


def _make_g6sum(D, OUTR, C):
    nc, ns = _sc_info()
    nw = nc * ns
    per_w = OUTR // nw
    steps = per_w // C
    assert steps % 2 == 0 and per_w % C == 0

    mesh = plsc.VectorSubcoreMesh(core_axis_name="c", subcore_axis_name="s")

    vpr = D // 16
    ng = 2
    gl = (MAX_NB // ng) * C

    scratch = [
        pltpu.VMEM((2, ng, gl), jnp.int32),
        pltpu.VMEM((2, ng, gl, D), jnp.float32),
        pltpu.VMEM((C, D), jnp.float32),
        pltpu.SemaphoreType.DMA,
        pltpu.SemaphoreType.DMA,
    ]

    def body(table, idxr, out, idx_v, gbuf, acc, sg0, sg1):
        wid = lax.axis_index("s") * nc + lax.axis_index("c")
        base = wid * per_w
        sg = [sg0, sg1]

        def fire_g(g):
            for h in range(ng):
                pltpu.async_copy(table.at[idx_v.at[g, h]], gbuf.at[g, h],
                                 sg[g])

        def wait_g(g):
            for h in range(ng):
                pltpu.make_async_copy(table.at[idx_v.at[g, h]],
                                      gbuf.at[g, h], sg[g]).wait()

        pltpu.sync_copy(idxr.at[wid, 0], idx_v.at[0])
        fire_g(0)

        def halfstep(s, g, g1):
            @pl.when(s + 1 < steps)
            def _():
                pltpu.sync_copy(idxr.at[wid, s + 1], idx_v.at[g1])

            wait_g(g)

            @pl.when(s + 1 < steps)
            def _():
                fire_g(g1)

            def row(i, c2):
                for v in range(vpr):
                    sl = pl.ds(v * 16, 16)
                    x = gbuf[g, 0, i, sl]
                    for h in range(ng):
                        for k in range(MAX_NB // ng):
                            if h == 0 and k == 0:
                                continue
                            x = x + gbuf[g, h, k * C + i, sl]
                    acc[i, sl] = x
                return c2

            lax.fori_loop(0, C, row, 0)
            pltpu.sync_copy(acc, out.at[pl.ds(base + s * C, C)])

        def body2(s2, carry):
            halfstep(2 * s2, 0, 1)
            halfstep(2 * s2 + 1, 1, 0)
            return carry

        lax.fori_loop(0, steps // 2, body2, 0)

    return pl.kernel(
        body,
        out_type=jax.ShapeDtypeStruct((OUTR, D), jnp.float32),
        mesh=mesh,
        scratch_types=scratch,
    )


def _make_gather_copy(D, OUTR):
    nc, ns = _sc_info()
    nw = nc * ns
    per_w = OUTR // nw
    steps = per_w // (_C * MAX_NB)

    mesh = plsc.VectorSubcoreMesh(core_axis_name="c", subcore_axis_name="s")

    scratch = [
        pltpu.VMEM((MAX_NB, _C), jnp.int32),
        pltpu.VMEM((MAX_NB * _C, D), jnp.float32),
        pltpu.SemaphoreType.DMA,
    ]

    def body_copy(table, idxr, out, idx_v, gbuf, sem):
        wid = lax.axis_index("s") * nc + lax.axis_index("c")
        base = wid * per_w

        def step(s, carry):
            pltpu.sync_copy(idxr.at[wid, s], idx_v)
            cps = [
                pltpu.async_copy(table.at[idx_v.at[k]],
                                 gbuf.at[pl.ds(k * _C, _C)], sem)
                for k in range(MAX_NB)
            ]
            for cp in cps:
                cp.wait()
            pltpu.sync_copy(
                gbuf, out.at[pl.ds(base + s * _C * MAX_NB, _C * MAX_NB)])
            return carry

        lax.fori_loop(0, steps, step, 0)

    return pl.kernel(
        body_copy,
        out_type=jax.ShapeDtypeStruct((OUTR, D), jnp.float32),
        mesh=mesh,
        scratch_types=scratch,
    )


def _pack_idx6(idx2d, outr, c):
    r = idx2d.shape[0]
    if r < outr:
        idx2d = jnp.pad(idx2d, ((0, outr - r), (0, 0)))
    per_w = outr // _NW
    s = per_w // c
    return idx2d.reshape(_NW, s, c, MAX_NB).transpose(0, 1, 3, 2).reshape(
        _NW, s, 2, (MAX_NB // 2) * c)


def _pack_idx1(idx1d, outr):
    r = idx1d.shape[0]
    if r < outr:
        idx1d = jnp.pad(idx1d, (0, outr - r))
    per_w = outr // _NW
    s = per_w // (_C * MAX_NB)
    return idx1d.reshape(_NW, s, MAX_NB, _C)


_TC_R = 1600


def _h0_body(fb_ref, w_ref, o_ref):
    o_ref[...] = jax.nn.relu(
        jnp.dot(fb_ref[...], w_ref[...], preferred_element_type=jnp.float32))


def _q0_body(h0_ref, am_ref, wh_ref, wa_ref, o_ref):
    o_ref[...] = (
        jnp.dot(h0_ref[...], wh_ref[...], preferred_element_type=jnp.float32)
        + jnp.dot(am_ref[...], wa_ref[...], preferred_element_type=jnp.float32))


def _q_body(h0_ref, s_ref, am_ref, wh_ref, wa_ref, o_ref):
    h = jax.nn.relu(h0_ref[...] + s_ref[...])
    o_ref[...] = (
        jnp.dot(h, wh_ref[...], preferred_element_type=jnp.float32)
        + jnp.dot(am_ref[...], wa_ref[...], preferred_element_type=jnp.float32))


def _h3_body(h0_ref, s_ref, o_ref):
    o_ref[...] = jax.nn.relu(h0_ref[...] + s_ref[...])


def _out_body(sh_ref, sa_ref, vh_ref, va_ref, o_ref):
    dn = (((1,), (1,)), ((), ()))
    o_ref[...] = jax.nn.relu(
        lax.dot_general(vh_ref[...], sh_ref[...], dn,
                        preferred_element_type=jnp.float32)
        + lax.dot_general(va_ref[...], sa_ref[...], dn,
                          preferred_element_type=jnp.float32))


def _row_spec(d):
    return pl.BlockSpec((_TC_R, d), lambda i: (i, 0))


def _full_spec(shape):
    return pl.BlockSpec(shape, lambda i: (0, 0))


def kernel(fatoms, fbonds, W_ein, W_edge, W_eout, out_n, bgraph, aingraph):
    E, N = N_BONDS, N_ATOMS
    grid = (E // _TC_R,)
    params = pltpu.CompilerParams(dimension_semantics=("parallel",))

    fb16 = jnp.pad(fbonds, ((0, 0), (0, 16 - fbonds.shape[1])))
    wein_t = jnp.pad(W_ein.T, ((0, 16 - W_ein.shape[1]), (0, 0)))
    W_h = W_edge[:, :HIDDEN]
    W_a = W_edge[:, HIDDEN:]
    V_h = W_eout[:, :HIDDEN]
    V_a = W_eout[:, HIDDEN:]
    wa_t = jnp.pad(W_a.T, ((0, _AF_PAD - ATOM_FDIM), (0, 0)))
    va64 = jnp.pad(V_a, ((0, 0), (0, _AF_PAD - ATOM_FDIM)))
    fat64 = jnp.pad(
        fatoms, ((0, _AT_ROWS - N), (0, _AF_PAD - ATOM_FDIM)))
    idx_full = jnp.concatenate(
        [jnp.full((1,), N, jnp.int32), out_n.astype(jnp.int32)])

    am_rows = 161280
    idx_am = _pack_idx1(idx_full, am_rows)
    bg_rows = 163840
    c_bg, c_ain = 32, 32
    idx_bg = _pack_idx6(bgraph.astype(jnp.int32), bg_rows, c_bg)
    ain_rows = 10240
    idx_ain = _pack_idx6(aingraph.astype(jnp.int32), ain_rows, c_ain)

    h0 = pl.pallas_call(
        _h0_body,
        grid=grid,
        in_specs=[_row_spec(16), _full_spec((16, HIDDEN))],
        out_specs=_row_spec(HIDDEN),
        out_shape=jax.ShapeDtypeStruct((E, HIDDEN), jnp.float32),
        compiler_params=params,
    )(fb16, wein_t)

    amraw_pad = _make_gather_copy(_AF_PAD, am_rows)(fat64, idx_am)

    g6_bond = _make_g6sum(HIDDEN, bg_rows, c_bg)
    q_call = pl.pallas_call(
        _q_body,
        grid=grid,
        in_specs=[_row_spec(HIDDEN), _row_spec(HIDDEN), _row_spec(_AF_PAD),
                  _full_spec((HIDDEN, HIDDEN)), _full_spec((_AF_PAD, HIDDEN))],
        out_specs=_row_spec(HIDDEN),
        out_shape=jax.ShapeDtypeStruct((E, HIDDEN), jnp.float32),
        compiler_params=params,
    )
    q0_call = pl.pallas_call(
        _q0_body,
        grid=grid,
        in_specs=[_row_spec(HIDDEN), _row_spec(_AF_PAD),
                  _full_spec((HIDDEN, HIDDEN)), _full_spec((_AF_PAD, HIDDEN))],
        out_specs=_row_spec(HIDDEN),
        out_shape=jax.ShapeDtypeStruct((E, HIDDEN), jnp.float32),
        compiler_params=params,
    )
    amraw = amraw_pad[:E]
    q = q0_call(h0, amraw, W_h.T, wa_t)
    for _ in range(DEPTH - 1):
        s = g6_bond(q, idx_bg)[:E]
        q = q_call(h0, s, amraw, W_h.T, wa_t)
    s = g6_bond(q, idx_bg)[:E]

    h3 = pl.pallas_call(
        _h3_body,
        grid=grid,
        in_specs=[_row_spec(HIDDEN), _row_spec(HIDDEN)],
        out_specs=_row_spec(HIDDEN),
        out_shape=jax.ShapeDtypeStruct((E, HIDDEN), jnp.float32),
        compiler_params=params,
    )(h0, s)

    s_h = _make_g6sum(HIDDEN, ain_rows, c_ain)(h3, idx_ain)
    s_a = _make_g6sum(_AF_PAD, ain_rows, c_ain)(amraw_pad, idx_ain)

    out = pl.pallas_call(
        _out_body,
        grid=(1,),
        in_specs=[_full_spec((N, HIDDEN)), _full_spec((N, _AF_PAD)),
                  _full_spec((OUT, HIDDEN)), _full_spec((OUT, _AF_PAD))],
        out_specs=_full_spec((OUT, N)),
        out_shape=jax.ShapeDtypeStruct((OUT, N), jnp.float32),
        compiler_params=params,
    )(s_h, s_a, V_h, va64)
    return out


# speedup vs baseline: 1.8753x; 1.8753x over previous
import functools

import jax
import jax.numpy as jnp
from jax import lax
from jax.experimental import pallas as pl
from jax.experimental.pallas import tpu as pltpu
from jax.experimental.pallas import tpu_sc as plsc

ATOM_FDIM = 39
N_ATOMS = 10000
N_BONDS = 160000
MAX_NB = 6
HIDDEN = 256
OUT = 256
DEPTH = 3

_AF_PAD = 128
_AT_ROWS = N_ATOMS + 8

_NW = 32
_C = 40


def _sc_info():
    info = plsc.get_sparse_core_info()
    return info.num_cores, info.num_subcores


def _make_g6sum(D, OUTR, C, dtype=jnp.float32):
    nc, ns = _sc_info()
    nw = nc * ns
    per_w = OUTR // nw
    steps = per_w // C
    assert per_w % C == 0

    mesh = plsc.VectorSubcoreMesh(core_axis_name="c", subcore_axis_name="s")

    f32 = dtype == jnp.float32
    ng = 2
    gl = (MAX_NB // ng) * C
    assert C % 2 == 0

    scratch = [
        pltpu.VMEM((ng, gl), jnp.int32),
        pltpu.VMEM((ng, gl, D), dtype),
        pltpu.VMEM((C, D), dtype),
        pltpu.SemaphoreType.DMA,
    ]

    def body(table, idxr, out, idx_v, gbuf, acc, sem):
        wid = lax.axis_index("s") * nc + lax.axis_index("c")
        base = wid * per_w

        def step(s, carry):
            pltpu.sync_copy(idxr.at[wid, s], idx_v)
            cps = [
                pltpu.async_copy(table.at[idx_v.at[h]], gbuf.at[h], sem)
                for h in range(ng)
            ]
            for cp in cps:
                cp.wait()

            def row(i, c2):
                for v in range(D // 16):
                    sl = pl.ds(v * 16, 16)
                    x = gbuf[0, i, sl]
                    for h in range(ng):
                        for k in range(MAX_NB // ng):
                            if h == 0 and k == 0:
                                continue
                            x = x + gbuf[h, k * C + i, sl]
                    acc[i, sl] = x
                return c2

            def rowpair(j, c2):
                i = pl.multiple_of(2 * j, 2)
                for v in range(D // 16):
                    sl = pl.ds(v * 16, 16)
                    x = gbuf[0, pl.ds(i, 2), sl]
                    for h in range(ng):
                        for k in range(MAX_NB // ng):
                            if h == 0 and k == 0:
                                continue
                            r = pl.multiple_of(k * C + i, 2)
                            x = x + gbuf[h, pl.ds(r, 2), sl]
                    acc[pl.ds(i, 2), sl] = x
                return c2

            if f32:
                lax.fori_loop(0, C, row, 0)
            else:
                lax.fori_loop(0, C // 2, rowpair, 0)
            pltpu.sync_copy(acc, out.at[pl.ds(base + s * C, C)])
            return carry

        lax.fori_loop(0, steps, step, 0)

    return pl.kernel(
        body,
        out_type=jax.ShapeDtypeStruct((OUTR, D), dtype),
        mesh=mesh,
        scratch_types=scratch,
    )


def _make_gather_copy(D, OUTR):
    nc, ns = _sc_info()
    nw = nc * ns
    per_w = OUTR // nw
    steps = per_w // (_C * MAX_NB)

    mesh = plsc.VectorSubcoreMesh(core_axis_name="c", subcore_axis_name="s")

    scratch = [
        pltpu.VMEM((MAX_NB, _C), jnp.int32),
        pltpu.VMEM((MAX_NB * _C, D), jnp.float32),
        pltpu.SemaphoreType.DMA,
    ]

    def body_copy(table, idxr, out, idx_v, gbuf, sem):
        wid = lax.axis_index("s") * nc + lax.axis_index("c")
        base = wid * per_w

        def step(s, carry):
            pltpu.sync_copy(idxr.at[wid, s], idx_v)
            cps = [
                pltpu.async_copy(table.at[idx_v.at[k]],
                                 gbuf.at[pl.ds(k * _C, _C)], sem)
                for k in range(MAX_NB)
            ]
            for cp in cps:
                cp.wait()
            pltpu.sync_copy(
                gbuf, out.at[pl.ds(base + s * _C * MAX_NB, _C * MAX_NB)])
            return carry

        lax.fori_loop(0, steps, step, 0)

    return pl.kernel(
        body_copy,
        out_type=jax.ShapeDtypeStruct((OUTR, D), jnp.float32),
        mesh=mesh,
        scratch_types=scratch,
    )


def _pack_idx6(idx2d, outr, c):
    r = idx2d.shape[0]
    if r < outr:
        idx2d = jnp.pad(idx2d, ((0, outr - r), (0, 0)))
    per_w = outr // _NW
    s = per_w // c
    return idx2d.reshape(_NW, s, c, MAX_NB).transpose(0, 1, 3, 2).reshape(
        _NW, s, 2, (MAX_NB // 2) * c)


def _pack_idx1(idx1d, outr):
    r = idx1d.shape[0]
    if r < outr:
        idx1d = jnp.pad(idx1d, (0, outr - r))
    per_w = outr // _NW
    s = per_w // (_C * MAX_NB)
    return idx1d.reshape(_NW, s, MAX_NB, _C)


_TC_R = 1600


def _h0_body(fb_ref, w_ref, o_ref):
    o_ref[...] = jax.nn.relu(
        jnp.dot(fb_ref[...], w_ref[...], preferred_element_type=jnp.float32))


def _q0_body(h0_ref, am_ref, wh_ref, wa_ref, o_ref):
    o_ref[...] = (
        jnp.dot(h0_ref[...], wh_ref[...], preferred_element_type=jnp.float32)
        + jnp.dot(am_ref[...], wa_ref[...], preferred_element_type=jnp.float32)
    ).astype(o_ref.dtype)


def _q_body(h0_ref, s_ref, am_ref, wh_ref, wa_ref, o_ref):
    h = jax.nn.relu(h0_ref[...] + s_ref[...].astype(jnp.float32))
    o_ref[...] = (
        jnp.dot(h, wh_ref[...], preferred_element_type=jnp.float32)
        + jnp.dot(am_ref[...], wa_ref[...], preferred_element_type=jnp.float32)
    ).astype(o_ref.dtype)


def _h3_body(h0_ref, s_ref, o_ref):
    o_ref[...] = jax.nn.relu(
        h0_ref[...] + s_ref[...].astype(jnp.float32)).astype(o_ref.dtype)


def _out_body(sh_ref, sa_ref, vh_ref, va_ref, o_ref):
    dn = (((1,), (1,)), ((), ()))
    o_ref[...] = jax.nn.relu(
        lax.dot_general(vh_ref[...], sh_ref[...].astype(jnp.float32), dn,
                        preferred_element_type=jnp.float32)
        + lax.dot_general(va_ref[...], sa_ref[...], dn,
                          preferred_element_type=jnp.float32))


def _row_spec(d):
    return pl.BlockSpec((_TC_R, d), lambda i: (i, 0))


def _full_spec(shape):
    return pl.BlockSpec(shape, lambda i: (0, 0))


def kernel(fatoms, fbonds, W_ein, W_edge, W_eout, out_n, bgraph, aingraph):
    E, N = N_BONDS, N_ATOMS
    grid = (E // _TC_R,)
    params = pltpu.CompilerParams(dimension_semantics=("parallel",))

    fb16 = jnp.pad(fbonds, ((0, 0), (0, 16 - fbonds.shape[1])))
    wein_t = jnp.pad(W_ein.T, ((0, 16 - W_ein.shape[1]), (0, 0)))
    W_h = W_edge[:, :HIDDEN]
    W_a = W_edge[:, HIDDEN:]
    V_h = W_eout[:, :HIDDEN]
    V_a = W_eout[:, HIDDEN:]
    wa_t = jnp.pad(W_a.T, ((0, _AF_PAD - ATOM_FDIM), (0, 0)))
    va64 = jnp.pad(V_a, ((0, 0), (0, _AF_PAD - ATOM_FDIM)))
    fat64 = jnp.pad(
        fatoms, ((0, _AT_ROWS - N), (0, _AF_PAD - ATOM_FDIM)))
    idx_full = jnp.concatenate(
        [jnp.full((1,), N, jnp.int32), out_n.astype(jnp.int32)])

    am_rows = 161280
    idx_am = _pack_idx1(idx_full, am_rows)
    bg_rows = E
    c_bg, c_ain = 40, 32
    idx_bg = _pack_idx6(bgraph.astype(jnp.int32), bg_rows, c_bg)
    ain_rows = 10240
    idx_ain = _pack_idx6(aingraph.astype(jnp.int32), ain_rows, c_ain)

    h0 = pl.pallas_call(
        _h0_body,
        grid=grid,
        in_specs=[_row_spec(16), _full_spec((16, HIDDEN))],
        out_specs=_row_spec(HIDDEN),
        out_shape=jax.ShapeDtypeStruct((E, HIDDEN), jnp.float32),
        compiler_params=params,
    )(fb16, wein_t)

    amraw_pad = _make_gather_copy(_AF_PAD, am_rows)(fat64, idx_am)

    g6_bond = _make_g6sum(HIDDEN, bg_rows, c_bg)
    q_call = pl.pallas_call(
        _q_body,
        grid=grid,
        in_specs=[_row_spec(HIDDEN), _row_spec(HIDDEN), _row_spec(_AF_PAD),
                  _full_spec((HIDDEN, HIDDEN)), _full_spec((_AF_PAD, HIDDEN))],
        out_specs=_row_spec(HIDDEN),
        out_shape=jax.ShapeDtypeStruct((E, HIDDEN), jnp.float32),
        compiler_params=params,
    )
    q0_call = pl.pallas_call(
        _q0_body,
        grid=grid,
        in_specs=[_row_spec(HIDDEN), _row_spec(_AF_PAD),
                  _full_spec((HIDDEN, HIDDEN)), _full_spec((_AF_PAD, HIDDEN))],
        out_specs=_row_spec(HIDDEN),
        out_shape=jax.ShapeDtypeStruct((E, HIDDEN), jnp.float32),
        compiler_params=params,
    )
    amraw = amraw_pad[:E]
    q = q0_call(h0, amraw, W_h.T, wa_t)
    for _ in range(DEPTH - 1):
        s = g6_bond(q, idx_bg)
        q = q_call(h0, s, amraw, W_h.T, wa_t)
    s = g6_bond(q, idx_bg)

    h3 = pl.pallas_call(
        _h3_body,
        grid=grid,
        in_specs=[_row_spec(HIDDEN), _row_spec(HIDDEN)],
        out_specs=_row_spec(HIDDEN),
        out_shape=jax.ShapeDtypeStruct((E, HIDDEN), jnp.float32),
        compiler_params=params,
    )(h0, s)

    s_h = _make_g6sum(HIDDEN, ain_rows, c_ain)(h3, idx_ain)
    s_a = _make_g6sum(_AF_PAD, ain_rows, c_ain)(amraw_pad, idx_ain)

    out = pl.pallas_call(
        _out_body,
        grid=(1,),
        in_specs=[_full_spec((N, HIDDEN)), _full_spec((N, _AF_PAD)),
                  _full_spec((OUT, HIDDEN)), _full_spec((OUT, _AF_PAD))],
        out_specs=_full_spec((OUT, N)),
        out_shape=jax.ShapeDtypeStruct((OUT, N), jnp.float32),
        compiler_params=params,
    )(s_h, s_a, V_h, va64)
    return out


# speedup vs baseline: 2.1954x; 1.1707x over previous
import functools

import jax
import jax.numpy as jnp
from jax import lax
from jax.experimental import pallas as pl
from jax.experimental.pallas import tpu as pltpu
from jax.experimental.pallas import tpu_sc as plsc

ATOM_FDIM = 39
N_ATOMS = 10000
N_BONDS = 160000
MAX_NB = 6
HIDDEN = 256
OUT = 256
DEPTH = 3

_AF_PAD = 128
_AT_ROWS = N_ATOMS + 8

_NW = 32
_C = 40


def _sc_info():
    info = plsc.get_sparse_core_info()
    return info.num_cores, info.num_subcores


def _make_g6sum(D, OUTR, C, dtype=jnp.float32, pipelined=False):
    nc, ns = _sc_info()
    nw = nc * ns
    per_w = OUTR // nw
    steps = per_w // C
    assert per_w % C == 0

    mesh = plsc.VectorSubcoreMesh(core_axis_name="c", subcore_axis_name="s")

    f32 = dtype == jnp.float32
    ng = 2
    gl = (MAX_NB // ng) * C
    assert C % 2 == 0

    nbuf = 2 if pipelined else 1
    assert C % 8 == 0

    scratch = [
        pltpu.VMEM((nbuf, ng, gl), jnp.int32),
        pltpu.VMEM((nbuf, ng, gl, D), dtype),
        pltpu.SemaphoreType.DMA,
        pltpu.SemaphoreType.DMA,
    ]

    def body(table, idxr, out, idx_v, gbuf, sem, sem1):
        wid = lax.axis_index("s") * nc + lax.axis_index("c")
        base = wid * per_w
        sg = [sem, sem1]

        def run_sum(g):
            def row(i, c2):
                for v in range(D // 16):
                    sl = pl.ds(v * 16, 16)
                    x = gbuf[g, 0, i, sl]
                    for h in range(ng):
                        for k in range(MAX_NB // ng):
                            if h == 0 and k == 0:
                                continue
                            x = x + gbuf[g, h, k * C + i, sl]
                    gbuf[g, 0, i, sl] = x
                return c2

            def rowpair(j, c2):
                i = pl.multiple_of(2 * j, 2)
                for v in range(D // 16):
                    sl = pl.ds(v * 16, 16)
                    x = gbuf[g, 0, pl.ds(i, 2), sl]
                    for h in range(ng):
                        for k in range(MAX_NB // ng):
                            if h == 0 and k == 0:
                                continue
                            r = pl.multiple_of(k * C + i, 2)
                            x = x + gbuf[g, h, pl.ds(r, 2), sl]
                    gbuf[g, 0, pl.ds(i, 2), sl] = x
                return c2

            if f32:
                lax.fori_loop(0, C, row, 0)
            else:
                lax.fori_loop(0, C // 2, rowpair, 0)

        def fire_g(g):
            for h in range(ng):
                pltpu.async_copy(table.at[idx_v.at[g, h]], gbuf.at[g, h],
                                 sg[g])

        def wait_g(g):
            for h in range(ng):
                pltpu.make_async_copy(table.at[idx_v.at[g, h]],
                                      gbuf.at[g, h], sg[g]).wait()

        def copy_out(s, g):
            pltpu.sync_copy(gbuf.at[g, 0, pl.ds(0, C)],
                            out.at[pl.ds(base + s * C, C)])

        if not pipelined:
            def step(s, carry):
                pltpu.sync_copy(idxr.at[wid, s], idx_v.at[0])
                fire_g(0)
                wait_g(0)
                run_sum(0)
                copy_out(s, 0)
                return carry

            lax.fori_loop(0, steps, step, 0)
        else:
            pltpu.sync_copy(idxr.at[wid, 0], idx_v.at[0])
            fire_g(0)

            def halfstep(s, g, g1):
                @pl.when(s + 1 < steps)
                def _():
                    pltpu.sync_copy(idxr.at[wid, s + 1], idx_v.at[g1])

                wait_g(g)

                @pl.when(s + 1 < steps)
                def _():
                    fire_g(g1)

                run_sum(g)
                copy_out(s, g)

            first = steps % 2
            if first:
                halfstep(0, 0, 1)

            def body2(s2, carry):
                halfstep(2 * s2 + first, first, 1 - first)
                halfstep(2 * s2 + 1 + first, 1 - first, first)
                return carry

            lax.fori_loop(0, steps // 2, body2, 0)

    return pl.kernel(
        body,
        out_type=jax.ShapeDtypeStruct((OUTR, D), dtype),
        mesh=mesh,
        scratch_types=scratch,
    )


def _make_gather_copy(D, OUTR):
    nc, ns = _sc_info()
    nw = nc * ns
    per_w = OUTR // nw
    steps = per_w // (_C * MAX_NB)

    mesh = plsc.VectorSubcoreMesh(core_axis_name="c", subcore_axis_name="s")

    scratch = [
        pltpu.VMEM((MAX_NB, _C), jnp.int32),
        pltpu.VMEM((MAX_NB * _C, D), jnp.float32),
        pltpu.SemaphoreType.DMA,
    ]

    def body_copy(table, idxr, out, idx_v, gbuf, sem):
        wid = lax.axis_index("s") * nc + lax.axis_index("c")
        base = wid * per_w

        def step(s, carry):
            pltpu.sync_copy(idxr.at[wid, s], idx_v)
            cps = [
                pltpu.async_copy(table.at[idx_v.at[k]],
                                 gbuf.at[pl.ds(k * _C, _C)], sem)
                for k in range(MAX_NB)
            ]
            for cp in cps:
                cp.wait()
            pltpu.sync_copy(
                gbuf, out.at[pl.ds(base + s * _C * MAX_NB, _C * MAX_NB)])
            return carry

        lax.fori_loop(0, steps, step, 0)

    return pl.kernel(
        body_copy,
        out_type=jax.ShapeDtypeStruct((OUTR, D), jnp.float32),
        mesh=mesh,
        scratch_types=scratch,
    )


def _pack_idx6(idx2d, outr, c):
    r = idx2d.shape[0]
    if r < outr:
        idx2d = jnp.pad(idx2d, ((0, outr - r), (0, 0)))
    per_w = outr // _NW
    s = per_w // c
    return idx2d.reshape(_NW, s, c, MAX_NB).transpose(0, 1, 3, 2).reshape(
        _NW, s, 2, (MAX_NB // 2) * c)


def _pack_idx1(idx1d, outr):
    r = idx1d.shape[0]
    if r < outr:
        idx1d = jnp.pad(idx1d, (0, outr - r))
    per_w = outr // _NW
    s = per_w // (_C * MAX_NB)
    return idx1d.reshape(_NW, s, MAX_NB, _C)


_TC_R = 1600


def _h0_body(fb_ref, w_ref, o_ref):
    o_ref[...] = jax.nn.relu(
        jnp.dot(fb_ref[...], w_ref[...], preferred_element_type=jnp.float32))


def _q0_body(h0_ref, am_ref, wh_ref, wa_ref, o_ref):
    o_ref[...] = (
        jnp.dot(h0_ref[...], wh_ref[...], preferred_element_type=jnp.float32)
        + jnp.dot(am_ref[...], wa_ref[...], preferred_element_type=jnp.float32)
    ).astype(o_ref.dtype)


def _q_body(h0_ref, s_ref, am_ref, wh_ref, wa_ref, o_ref):
    h = jax.nn.relu(h0_ref[...] + s_ref[...].astype(jnp.float32))
    o_ref[...] = (
        jnp.dot(h, wh_ref[...], preferred_element_type=jnp.float32)
        + jnp.dot(am_ref[...], wa_ref[...], preferred_element_type=jnp.float32)
    ).astype(o_ref.dtype)


def _h3_body(h0_ref, s_ref, o_ref):
    o_ref[...] = jax.nn.relu(
        h0_ref[...] + s_ref[...].astype(jnp.float32)).astype(o_ref.dtype)


def _out_body(sh_ref, sa_ref, vh_ref, va_ref, o_ref):
    dn = (((1,), (1,)), ((), ()))
    o_ref[...] = jax.nn.relu(
        lax.dot_general(vh_ref[...], sh_ref[...].astype(jnp.float32), dn,
                        preferred_element_type=jnp.float32)
        + lax.dot_general(va_ref[...], sa_ref[...], dn,
                          preferred_element_type=jnp.float32))


def _row_spec(d):
    return pl.BlockSpec((_TC_R, d), lambda i: (i, 0))


def _full_spec(shape):
    return pl.BlockSpec(shape, lambda i: (0, 0))


def kernel(fatoms, fbonds, W_ein, W_edge, W_eout, out_n, bgraph, aingraph):
    E, N = N_BONDS, N_ATOMS
    grid = (E // _TC_R,)
    params = pltpu.CompilerParams(dimension_semantics=("parallel",))

    fb16 = jnp.pad(fbonds, ((0, 0), (0, 16 - fbonds.shape[1])))
    wein_t = jnp.pad(W_ein.T, ((0, 16 - W_ein.shape[1]), (0, 0)))
    W_h = W_edge[:, :HIDDEN]
    W_a = W_edge[:, HIDDEN:]
    V_h = W_eout[:, :HIDDEN]
    V_a = W_eout[:, HIDDEN:]
    wa_t = jnp.pad(W_a.T, ((0, _AF_PAD - ATOM_FDIM), (0, 0)))
    va64 = jnp.pad(V_a, ((0, 0), (0, _AF_PAD - ATOM_FDIM)))
    fat64 = jnp.pad(
        fatoms, ((0, _AT_ROWS - N), (0, _AF_PAD - ATOM_FDIM)))
    idx_full = jnp.concatenate(
        [jnp.full((1,), N, jnp.int32), out_n.astype(jnp.int32)])

    am_rows = 161280
    idx_am = _pack_idx1(idx_full, am_rows)
    bg_rows = E
    c_bg, c_ain = 40, 32
    idx_bg = _pack_idx6(bgraph.astype(jnp.int32), bg_rows, c_bg)
    ain_rows = 10240
    idx_ain = _pack_idx6(aingraph.astype(jnp.int32), ain_rows, c_ain)

    h0 = pl.pallas_call(
        _h0_body,
        grid=grid,
        in_specs=[_row_spec(16), _full_spec((16, HIDDEN))],
        out_specs=_row_spec(HIDDEN),
        out_shape=jax.ShapeDtypeStruct((E, HIDDEN), jnp.float32),
        compiler_params=params,
    )(fb16, wein_t)

    amraw_pad = _make_gather_copy(_AF_PAD, am_rows)(fat64, idx_am)

    g6_bond = _make_g6sum(HIDDEN, bg_rows, c_bg, pipelined=True)
    q_call = pl.pallas_call(
        _q_body,
        grid=grid,
        in_specs=[_row_spec(HIDDEN), _row_spec(HIDDEN), _row_spec(_AF_PAD),
                  _full_spec((HIDDEN, HIDDEN)), _full_spec((_AF_PAD, HIDDEN))],
        out_specs=_row_spec(HIDDEN),
        out_shape=jax.ShapeDtypeStruct((E, HIDDEN), jnp.float32),
        compiler_params=params,
    )
    q0_call = pl.pallas_call(
        _q0_body,
        grid=grid,
        in_specs=[_row_spec(HIDDEN), _row_spec(_AF_PAD),
                  _full_spec((HIDDEN, HIDDEN)), _full_spec((_AF_PAD, HIDDEN))],
        out_specs=_row_spec(HIDDEN),
        out_shape=jax.ShapeDtypeStruct((E, HIDDEN), jnp.float32),
        compiler_params=params,
    )
    amraw = amraw_pad[:E]
    q = q0_call(h0, amraw, W_h.T, wa_t)
    for _ in range(DEPTH - 1):
        s = g6_bond(q, idx_bg)
        q = q_call(h0, s, amraw, W_h.T, wa_t)
    s = g6_bond(q, idx_bg)

    h3 = pl.pallas_call(
        _h3_body,
        grid=grid,
        in_specs=[_row_spec(HIDDEN), _row_spec(HIDDEN)],
        out_specs=_row_spec(HIDDEN),
        out_shape=jax.ShapeDtypeStruct((E, HIDDEN), jnp.float32),
        compiler_params=params,
    )(h0, s)

    s_h = _make_g6sum(HIDDEN, ain_rows, c_ain)(h3, idx_ain)
    s_a = _make_g6sum(_AF_PAD, ain_rows, c_ain)(amraw_pad, idx_ain)

    out = pl.pallas_call(
        _out_body,
        grid=(1,),
        in_specs=[_full_spec((N, HIDDEN)), _full_spec((N, _AF_PAD)),
                  _full_spec((OUT, HIDDEN)), _full_spec((OUT, _AF_PAD))],
        out_specs=_full_spec((OUT, N)),
        out_shape=jax.ShapeDtypeStruct((OUT, N), jnp.float32),
        compiler_params=params,
    )(s_h, s_a, V_h, va64)
    return out


# speedup vs baseline: 2.2038x; 1.0038x over previous
import functools

import jax
import jax.numpy as jnp
from jax import lax
from jax.experimental import pallas as pl
from jax.experimental.pallas import tpu as pltpu
from jax.experimental.pallas import tpu_sc as plsc

ATOM_FDIM = 39
N_ATOMS = 10000
N_BONDS = 160000
MAX_NB = 6
HIDDEN = 256
OUT = 256
DEPTH = 3

_AF_PAD = 128
_AT_ROWS = N_ATOMS + 8

_NW = 32
_C = 40


def _sc_info():
    info = plsc.get_sparse_core_info()
    return info.num_cores, info.num_subcores


def _make_g6sum(D, OUTR, C, dtype=jnp.float32, pipelined=False):
    nc, ns = _sc_info()
    nw = nc * ns
    per_w = OUTR // nw
    steps = per_w // C
    assert per_w % C == 0

    mesh = plsc.VectorSubcoreMesh(core_axis_name="c", subcore_axis_name="s")

    f32 = dtype == jnp.float32
    ng = 2
    gl = (MAX_NB // ng) * C
    assert C % 2 == 0

    nbuf = 2 if pipelined else 1
    assert C % 8 == 0

    scratch = [
        pltpu.VMEM((nbuf, ng, gl), jnp.int32),
        pltpu.VMEM((nbuf, ng, gl, D), dtype),
        pltpu.SemaphoreType.DMA,
        pltpu.SemaphoreType.DMA,
    ]

    def body(table, idxr, out, idx_v, gbuf, sem, sem1):
        wid = lax.axis_index("s") * nc + lax.axis_index("c")
        base = wid * per_w
        sg = [sem, sem1]

        def run_sum(g):
            def row(i, c2):
                for v in range(D // 16):
                    sl = pl.ds(v * 16, 16)
                    x = gbuf[g, 0, i, sl]
                    for h in range(ng):
                        for k in range(MAX_NB // ng):
                            if h == 0 and k == 0:
                                continue
                            x = x + gbuf[g, h, k * C + i, sl]
                    gbuf[g, 0, i, sl] = x
                return c2

            def rowpair(j, c2):
                i = pl.multiple_of(2 * j, 2)
                for v in range(D // 16):
                    sl = pl.ds(v * 16, 16)
                    x = gbuf[g, 0, pl.ds(i, 2), sl]
                    for h in range(ng):
                        for k in range(MAX_NB // ng):
                            if h == 0 and k == 0:
                                continue
                            r = pl.multiple_of(k * C + i, 2)
                            x = x + gbuf[g, h, pl.ds(r, 2), sl]
                    gbuf[g, 0, pl.ds(i, 2), sl] = x
                return c2

            if f32:
                lax.fori_loop(0, C, row, 0)
            else:
                lax.fori_loop(0, C // 2, rowpair, 0)

        def fire_g(g):
            for h in range(ng):
                pltpu.async_copy(table.at[idx_v.at[g, h]], gbuf.at[g, h],
                                 sg[g])

        def wait_g(g):
            for h in range(ng):
                pltpu.make_async_copy(table.at[idx_v.at[g, h]],
                                      gbuf.at[g, h], sg[g]).wait()

        def copy_out(s, g):
            pltpu.sync_copy(gbuf.at[g, 0, pl.ds(0, C)],
                            out.at[pl.ds(base + s * C, C)])

        if not pipelined:
            def step(s, carry):
                pltpu.sync_copy(idxr.at[wid, s], idx_v.at[0])
                fire_g(0)
                wait_g(0)
                run_sum(0)
                copy_out(s, 0)
                return carry

            lax.fori_loop(0, steps, step, 0)
        else:
            pltpu.sync_copy(idxr.at[wid, 0], idx_v.at[0])
            fire_g(0)

            def halfstep(s, g, g1):
                @pl.when(s + 1 < steps)
                def _():
                    pltpu.sync_copy(idxr.at[wid, s + 1], idx_v.at[g1])

                wait_g(g)

                @pl.when(s + 1 < steps)
                def _():
                    fire_g(g1)

                run_sum(g)
                copy_out(s, g)

            first = steps % 2
            if first:
                halfstep(0, 0, 1)

            def body2(s2, carry):
                halfstep(2 * s2 + first, first, 1 - first)
                halfstep(2 * s2 + 1 + first, 1 - first, first)
                return carry

            lax.fori_loop(0, steps // 2, body2, 0)

    return pl.kernel(
        body,
        out_type=jax.ShapeDtypeStruct((OUTR, D), dtype),
        mesh=mesh,
        scratch_types=scratch,
    )


def _make_gather_copy(D, OUTR):
    nc, ns = _sc_info()
    nw = nc * ns
    per_w = OUTR // nw
    steps = per_w // (_C * MAX_NB)

    mesh = plsc.VectorSubcoreMesh(core_axis_name="c", subcore_axis_name="s")

    scratch = [
        pltpu.VMEM((2, MAX_NB, _C), jnp.int32),
        pltpu.VMEM((2, MAX_NB * _C, D), jnp.float32),
        pltpu.SemaphoreType.DMA,
        pltpu.SemaphoreType.DMA,
    ]

    def body_copy(table, idxr, out, idx_v, gbuf, sem, sem1):
        wid = lax.axis_index("s") * nc + lax.axis_index("c")
        base = wid * per_w
        sg = [sem, sem1]

        def fire_g(g):
            for k in range(MAX_NB):
                pltpu.async_copy(table.at[idx_v.at[g, k]],
                                 gbuf.at[g, pl.ds(k * _C, _C)], sg[g])

        def wait_g(g):
            for k in range(MAX_NB):
                pltpu.make_async_copy(
                    table.at[idx_v.at[g, k]],
                    gbuf.at[g, pl.ds(k * _C, _C)], sg[g]).wait()

        pltpu.sync_copy(idxr.at[wid, 0], idx_v.at[0])
        fire_g(0)

        def halfstep(s, g, g1):
            @pl.when(s + 1 < steps)
            def _():
                pltpu.sync_copy(idxr.at[wid, s + 1], idx_v.at[g1])

            wait_g(g)

            @pl.when(s + 1 < steps)
            def _():
                fire_g(g1)

            pltpu.sync_copy(
                gbuf.at[g],
                out.at[pl.ds(base + s * _C * MAX_NB, _C * MAX_NB)])

        first = steps % 2
        if first:
            halfstep(0, 0, 1)

        def body2(s2, carry):
            halfstep(2 * s2 + first, first, 1 - first)
            halfstep(2 * s2 + 1 + first, 1 - first, first)
            return carry

        lax.fori_loop(0, steps // 2, body2, 0)

    return pl.kernel(
        body_copy,
        out_type=jax.ShapeDtypeStruct((OUTR, D), jnp.float32),
        mesh=mesh,
        scratch_types=scratch,
    )


def _pack_idx6(idx2d, outr, c):
    r = idx2d.shape[0]
    if r < outr:
        idx2d = jnp.pad(idx2d, ((0, outr - r), (0, 0)))
    per_w = outr // _NW
    s = per_w // c
    return idx2d.reshape(_NW, s, c, MAX_NB).transpose(0, 1, 3, 2).reshape(
        _NW, s, 2, (MAX_NB // 2) * c)


def _pack_idx1(idx1d, outr):
    r = idx1d.shape[0]
    if r < outr:
        idx1d = jnp.pad(idx1d, (0, outr - r))
    per_w = outr // _NW
    s = per_w // (_C * MAX_NB)
    return idx1d.reshape(_NW, s, MAX_NB, _C)


_TC_R = 1600


def _h0_body(fb_ref, w_ref, o_ref):
    o_ref[...] = jax.nn.relu(
        jnp.dot(fb_ref[...], w_ref[...], preferred_element_type=jnp.float32))


def _q0_body(h0_ref, am_ref, wh_ref, wa_ref, o_ref):
    o_ref[...] = (
        jnp.dot(h0_ref[...], wh_ref[...], preferred_element_type=jnp.float32)
        + jnp.dot(am_ref[...], wa_ref[...], preferred_element_type=jnp.float32)
    ).astype(o_ref.dtype)


def _q_body(h0_ref, s_ref, am_ref, wh_ref, wa_ref, o_ref):
    h = jax.nn.relu(h0_ref[...] + s_ref[...].astype(jnp.float32))
    o_ref[...] = (
        jnp.dot(h, wh_ref[...], preferred_element_type=jnp.float32)
        + jnp.dot(am_ref[...], wa_ref[...], preferred_element_type=jnp.float32)
    ).astype(o_ref.dtype)


def _h3_body(h0_ref, s_ref, o_ref):
    o_ref[...] = jax.nn.relu(
        h0_ref[...] + s_ref[...].astype(jnp.float32)).astype(o_ref.dtype)


def _out_body(sh_ref, sa_ref, vh_ref, va_ref, o_ref):
    dn = (((1,), (1,)), ((), ()))
    o_ref[...] = jax.nn.relu(
        lax.dot_general(vh_ref[...], sh_ref[...].astype(jnp.float32), dn,
                        preferred_element_type=jnp.float32)
        + lax.dot_general(va_ref[...], sa_ref[...], dn,
                          preferred_element_type=jnp.float32))


def _row_spec(d):
    return pl.BlockSpec((_TC_R, d), lambda i: (i, 0))


def _full_spec(shape):
    return pl.BlockSpec(shape, lambda i: (0, 0))


def kernel(fatoms, fbonds, W_ein, W_edge, W_eout, out_n, bgraph, aingraph):
    E, N = N_BONDS, N_ATOMS
    grid = (E // _TC_R,)
    params = pltpu.CompilerParams(dimension_semantics=("parallel",))

    fb16 = jnp.pad(fbonds, ((0, 0), (0, 16 - fbonds.shape[1])))
    wein_t = jnp.pad(W_ein.T, ((0, 16 - W_ein.shape[1]), (0, 0)))
    W_h = W_edge[:, :HIDDEN]
    W_a = W_edge[:, HIDDEN:]
    V_h = W_eout[:, :HIDDEN]
    V_a = W_eout[:, HIDDEN:]
    wa_t = jnp.pad(W_a.T, ((0, _AF_PAD - ATOM_FDIM), (0, 0)))
    va64 = jnp.pad(V_a, ((0, 0), (0, _AF_PAD - ATOM_FDIM)))
    fat64 = jnp.pad(
        fatoms, ((0, _AT_ROWS - N), (0, _AF_PAD - ATOM_FDIM)))
    idx_full = jnp.concatenate(
        [jnp.full((1,), N, jnp.int32), out_n.astype(jnp.int32)])

    am_rows = 161280
    idx_am = _pack_idx1(idx_full, am_rows)
    bg_rows = E
    c_bg, c_ain = 40, 32
    idx_bg = _pack_idx6(bgraph.astype(jnp.int32), bg_rows, c_bg)
    ain_rows = 10240
    idx_ain = _pack_idx6(aingraph.astype(jnp.int32), ain_rows, c_ain)

    h0 = pl.pallas_call(
        _h0_body,
        grid=grid,
        in_specs=[_row_spec(16), _full_spec((16, HIDDEN))],
        out_specs=_row_spec(HIDDEN),
        out_shape=jax.ShapeDtypeStruct((E, HIDDEN), jnp.float32),
        compiler_params=params,
    )(fb16, wein_t)

    amraw_pad = _make_gather_copy(_AF_PAD, am_rows)(fat64, idx_am)

    g6_bond = _make_g6sum(HIDDEN, bg_rows, c_bg, pipelined=True)
    q_call = pl.pallas_call(
        _q_body,
        grid=grid,
        in_specs=[_row_spec(HIDDEN), _row_spec(HIDDEN), _row_spec(_AF_PAD),
                  _full_spec((HIDDEN, HIDDEN)), _full_spec((_AF_PAD, HIDDEN))],
        out_specs=_row_spec(HIDDEN),
        out_shape=jax.ShapeDtypeStruct((E, HIDDEN), jnp.float32),
        compiler_params=params,
    )
    q0_call = pl.pallas_call(
        _q0_body,
        grid=grid,
        in_specs=[_row_spec(HIDDEN), _row_spec(_AF_PAD),
                  _full_spec((HIDDEN, HIDDEN)), _full_spec((_AF_PAD, HIDDEN))],
        out_specs=_row_spec(HIDDEN),
        out_shape=jax.ShapeDtypeStruct((E, HIDDEN), jnp.float32),
        compiler_params=params,
    )
    amraw = amraw_pad[:E]
    q = q0_call(h0, amraw, W_h.T, wa_t)
    for _ in range(DEPTH - 1):
        s = g6_bond(q, idx_bg)
        q = q_call(h0, s, amraw, W_h.T, wa_t)
    s = g6_bond(q, idx_bg)

    h3 = pl.pallas_call(
        _h3_body,
        grid=grid,
        in_specs=[_row_spec(HIDDEN), _row_spec(HIDDEN)],
        out_specs=_row_spec(HIDDEN),
        out_shape=jax.ShapeDtypeStruct((E, HIDDEN), jnp.float32),
        compiler_params=params,
    )(h0, s)

    s_h = _make_g6sum(HIDDEN, ain_rows, c_ain, pipelined=True)(h3, idx_ain)
    s_a = _make_g6sum(_AF_PAD, ain_rows, c_ain, pipelined=True)(
        amraw_pad, idx_ain)

    out = pl.pallas_call(
        _out_body,
        grid=(1,),
        in_specs=[_full_spec((N, HIDDEN)), _full_spec((N, _AF_PAD)),
                  _full_spec((OUT, HIDDEN)), _full_spec((OUT, _AF_PAD))],
        out_specs=_full_spec((OUT, N)),
        out_shape=jax.ShapeDtypeStruct((OUT, N), jnp.float32),
        compiler_params=params,
    )(s_h, s_a, V_h, va64)
    return out


# speedup vs baseline: 2.2820x; 1.0355x over previous
import functools

import jax
import jax.numpy as jnp
from jax import lax
from jax.experimental import pallas as pl
from jax.experimental.pallas import tpu as pltpu
from jax.experimental.pallas import tpu_sc as plsc

ATOM_FDIM = 39
N_ATOMS = 10000
N_BONDS = 160000
MAX_NB = 6
HIDDEN = 256
OUT = 256
DEPTH = 3

_AF_PAD = 128
_AT_ROWS = N_ATOMS + 8

_NW = 32
_C = 40


def _sc_info():
    info = plsc.get_sparse_core_info()
    return info.num_cores, info.num_subcores


def _make_g6sum(D, OUTR, C, dtype=jnp.float32, pipelined=False):
    nc, ns = _sc_info()
    nw = nc * ns
    per_w = OUTR // nw
    steps = per_w // C
    assert per_w % C == 0

    mesh = plsc.VectorSubcoreMesh(core_axis_name="c", subcore_axis_name="s")

    f32 = dtype == jnp.float32
    ng = 2
    gl = (MAX_NB // ng) * C
    assert C % 2 == 0

    nbuf = 2 if pipelined else 1
    assert C % 8 == 0

    scratch = [
        pltpu.VMEM((nbuf, ng, gl), jnp.int32),
        pltpu.VMEM((nbuf, ng, gl, D), dtype),
        pltpu.SemaphoreType.DMA,
        pltpu.SemaphoreType.DMA,
    ]

    def body(table, idxr, out, idx_v, gbuf, sem, sem1):
        wid = lax.axis_index("s") * nc + lax.axis_index("c")
        base = wid * per_w
        sg = [sem, sem1]

        def run_sum(g):
            def row(i, c2):
                for v in range(D // 16):
                    sl = pl.ds(v * 16, 16)
                    x = gbuf[g, 0, i, sl]
                    for h in range(ng):
                        for k in range(MAX_NB // ng):
                            if h == 0 and k == 0:
                                continue
                            x = x + gbuf[g, h, k * C + i, sl]
                    gbuf[g, 0, i, sl] = x
                return c2

            def rowpair(j, c2):
                i = pl.multiple_of(2 * j, 2)
                for v in range(D // 16):
                    sl = pl.ds(v * 16, 16)
                    x = gbuf[g, 0, pl.ds(i, 2), sl]
                    for h in range(ng):
                        for k in range(MAX_NB // ng):
                            if h == 0 and k == 0:
                                continue
                            r = pl.multiple_of(k * C + i, 2)
                            x = x + gbuf[g, h, pl.ds(r, 2), sl]
                    gbuf[g, 0, pl.ds(i, 2), sl] = x
                return c2

            if f32:
                lax.fori_loop(0, C, row, 0)
            else:
                lax.fori_loop(0, C // 2, rowpair, 0)

        def fire_g(g):
            for h in range(ng):
                pltpu.async_copy(table.at[idx_v.at[g, h]], gbuf.at[g, h],
                                 sg[g])

        def wait_g(g):
            for h in range(ng):
                pltpu.make_async_copy(table.at[idx_v.at[g, h]],
                                      gbuf.at[g, h], sg[g]).wait()

        def copy_out(s, g):
            pltpu.sync_copy(gbuf.at[g, 0, pl.ds(0, C)],
                            out.at[pl.ds(base + s * C, C)])

        if not pipelined:
            def step(s, carry):
                pltpu.sync_copy(idxr.at[wid, s], idx_v.at[0])
                fire_g(0)
                wait_g(0)
                run_sum(0)
                copy_out(s, 0)
                return carry

            lax.fori_loop(0, steps, step, 0)
        else:
            pltpu.sync_copy(idxr.at[wid, 0], idx_v.at[0])
            fire_g(0)

            def halfstep(s, g, g1):
                @pl.when(s + 1 < steps)
                def _():
                    pltpu.sync_copy(idxr.at[wid, s + 1], idx_v.at[g1])

                wait_g(g)

                @pl.when(s + 1 < steps)
                def _():
                    fire_g(g1)

                run_sum(g)
                copy_out(s, g)

            first = steps % 2
            if first:
                halfstep(0, 0, 1)

            def body2(s2, carry):
                halfstep(2 * s2 + first, first, 1 - first)
                halfstep(2 * s2 + 1 + first, 1 - first, first)
                return carry

            lax.fori_loop(0, steps // 2, body2, 0)

    return pl.kernel(
        body,
        out_type=jax.ShapeDtypeStruct((OUTR, D), dtype),
        mesh=mesh,
        scratch_types=scratch,
    )


_CP_G = 100


def _make_gather_copy(D, OUTR):
    nc, ns = _sc_info()
    nw = nc * ns
    per_w = OUTR // nw
    rps = 2 * _CP_G
    steps = per_w // rps
    assert per_w % rps == 0

    mesh = plsc.VectorSubcoreMesh(core_axis_name="c", subcore_axis_name="s")

    scratch = [
        pltpu.VMEM((2, 2, _CP_G), jnp.int32),
        pltpu.VMEM((2, rps, D), jnp.float32),
        pltpu.SemaphoreType.DMA,
        pltpu.SemaphoreType.DMA,
    ]

    def body_copy(table, idxr, out, idx_v, gbuf, sem, sem1):
        wid = lax.axis_index("s") * nc + lax.axis_index("c")
        base = wid * per_w
        sg = [sem, sem1]

        def fire_g(g):
            for k in range(2):
                pltpu.async_copy(table.at[idx_v.at[g, k]],
                                 gbuf.at[g, pl.ds(k * _CP_G, _CP_G)], sg[g])

        def wait_g(g):
            for k in range(2):
                pltpu.make_async_copy(
                    table.at[idx_v.at[g, k]],
                    gbuf.at[g, pl.ds(k * _CP_G, _CP_G)], sg[g]).wait()

        pltpu.sync_copy(idxr.at[wid, 0], idx_v.at[0])
        fire_g(0)

        def halfstep(s, g, g1):
            @pl.when(s + 1 < steps)
            def _():
                pltpu.sync_copy(idxr.at[wid, s + 1], idx_v.at[g1])

            wait_g(g)

            @pl.when(s + 1 < steps)
            def _():
                fire_g(g1)

            pltpu.sync_copy(gbuf.at[g], out.at[pl.ds(base + s * rps, rps)])

        first = steps % 2
        if first:
            halfstep(0, 0, 1)

        def body2(s2, carry):
            halfstep(2 * s2 + first, first, 1 - first)
            halfstep(2 * s2 + 1 + first, 1 - first, first)
            return carry

        lax.fori_loop(0, steps // 2, body2, 0)

    return pl.kernel(
        body_copy,
        out_type=jax.ShapeDtypeStruct((OUTR, D), jnp.float32),
        mesh=mesh,
        scratch_types=scratch,
    )


def _pack_idx6(idx2d, outr, c):
    r = idx2d.shape[0]
    if r < outr:
        idx2d = jnp.pad(idx2d, ((0, outr - r), (0, 0)))
    per_w = outr // _NW
    s = per_w // c
    return idx2d.reshape(_NW, s, c, MAX_NB).transpose(0, 1, 3, 2).reshape(
        _NW, s, 2, (MAX_NB // 2) * c)


def _pack_idx1(idx1d, outr):
    r = idx1d.shape[0]
    if r < outr:
        idx1d = jnp.pad(idx1d, (0, outr - r))
    per_w = outr // _NW
    s = per_w // (2 * _CP_G)
    return idx1d.reshape(_NW, s, 2, _CP_G)


_TC_R = 1600


def _h0_body(fb_ref, w_ref, o_ref):
    o_ref[...] = jax.nn.relu(
        jnp.dot(fb_ref[...], w_ref[...], preferred_element_type=jnp.float32))


def _q0_body(h0_ref, am_ref, wh_ref, wa_ref, o_ref):
    o_ref[...] = (
        jnp.dot(h0_ref[...], wh_ref[...], preferred_element_type=jnp.float32)
        + jnp.dot(am_ref[...], wa_ref[...], preferred_element_type=jnp.float32)
    ).astype(o_ref.dtype)


def _q_body(h0_ref, s_ref, am_ref, wh_ref, wa_ref, o_ref):
    h = jax.nn.relu(h0_ref[...] + s_ref[...].astype(jnp.float32))
    o_ref[...] = (
        jnp.dot(h, wh_ref[...], preferred_element_type=jnp.float32)
        + jnp.dot(am_ref[...], wa_ref[...], preferred_element_type=jnp.float32)
    ).astype(o_ref.dtype)


def _h3_body(h0_ref, s_ref, o_ref):
    o_ref[...] = jax.nn.relu(
        h0_ref[...] + s_ref[...].astype(jnp.float32)).astype(o_ref.dtype)


def _out_body(sh_ref, sa_ref, vh_ref, va_ref, o_ref):
    dn = (((1,), (1,)), ((), ()))
    o_ref[...] = jax.nn.relu(
        lax.dot_general(vh_ref[...], sh_ref[...].astype(jnp.float32), dn,
                        preferred_element_type=jnp.float32)
        + lax.dot_general(va_ref[...], sa_ref[...], dn,
                          preferred_element_type=jnp.float32))


def _row_spec(d):
    return pl.BlockSpec((_TC_R, d), lambda i: (i, 0))


def _full_spec(shape):
    return pl.BlockSpec(shape, lambda i: (0, 0))


def kernel(fatoms, fbonds, W_ein, W_edge, W_eout, out_n, bgraph, aingraph):
    E, N = N_BONDS, N_ATOMS
    grid = (E // _TC_R,)
    params = pltpu.CompilerParams(dimension_semantics=("parallel",))

    fb16 = jnp.pad(fbonds, ((0, 0), (0, 16 - fbonds.shape[1])))
    wein_t = jnp.pad(W_ein.T, ((0, 16 - W_ein.shape[1]), (0, 0)))
    W_h = W_edge[:, :HIDDEN]
    W_a = W_edge[:, HIDDEN:]
    V_h = W_eout[:, :HIDDEN]
    V_a = W_eout[:, HIDDEN:]
    wa_t = jnp.pad(W_a.T, ((0, _AF_PAD - ATOM_FDIM), (0, 0)))
    va64 = jnp.pad(V_a, ((0, 0), (0, _AF_PAD - ATOM_FDIM)))
    fat64 = jnp.pad(
        fatoms, ((0, _AT_ROWS - N), (0, _AF_PAD - ATOM_FDIM)))
    idx_full = jnp.concatenate(
        [jnp.full((1,), N, jnp.int32), out_n.astype(jnp.int32)])

    am_rows = E
    idx_am = _pack_idx1(idx_full, am_rows)
    bg_rows = E
    c_bg, c_ain = 40, 32
    idx_bg = _pack_idx6(bgraph.astype(jnp.int32), bg_rows, c_bg)
    ain_rows = 10240
    idx_ain = _pack_idx6(aingraph.astype(jnp.int32), ain_rows, c_ain)

    h0 = pl.pallas_call(
        _h0_body,
        grid=grid,
        in_specs=[_row_spec(16), _full_spec((16, HIDDEN))],
        out_specs=_row_spec(HIDDEN),
        out_shape=jax.ShapeDtypeStruct((E, HIDDEN), jnp.float32),
        compiler_params=params,
    )(fb16, wein_t)

    amraw_pad = _make_gather_copy(_AF_PAD, am_rows)(fat64, idx_am)

    g6_bond = _make_g6sum(HIDDEN, bg_rows, c_bg, pipelined=True)
    q_call = pl.pallas_call(
        _q_body,
        grid=grid,
        in_specs=[_row_spec(HIDDEN), _row_spec(HIDDEN), _row_spec(_AF_PAD),
                  _full_spec((HIDDEN, HIDDEN)), _full_spec((_AF_PAD, HIDDEN))],
        out_specs=_row_spec(HIDDEN),
        out_shape=jax.ShapeDtypeStruct((E, HIDDEN), jnp.float32),
        compiler_params=params,
    )
    q0_call = pl.pallas_call(
        _q0_body,
        grid=grid,
        in_specs=[_row_spec(HIDDEN), _row_spec(_AF_PAD),
                  _full_spec((HIDDEN, HIDDEN)), _full_spec((_AF_PAD, HIDDEN))],
        out_specs=_row_spec(HIDDEN),
        out_shape=jax.ShapeDtypeStruct((E, HIDDEN), jnp.float32),
        compiler_params=params,
    )
    amraw = amraw_pad[:E]
    q = q0_call(h0, amraw, W_h.T, wa_t)
    for _ in range(DEPTH - 1):
        s = g6_bond(q, idx_bg)
        q = q_call(h0, s, amraw, W_h.T, wa_t)
    s = g6_bond(q, idx_bg)

    h3 = pl.pallas_call(
        _h3_body,
        grid=grid,
        in_specs=[_row_spec(HIDDEN), _row_spec(HIDDEN)],
        out_specs=_row_spec(HIDDEN),
        out_shape=jax.ShapeDtypeStruct((E, HIDDEN), jnp.float32),
        compiler_params=params,
    )(h0, s)

    s_h = _make_g6sum(HIDDEN, ain_rows, c_ain, pipelined=True)(h3, idx_ain)
    s_a = _make_g6sum(_AF_PAD, ain_rows, c_ain, pipelined=True)(
        amraw_pad, idx_ain)

    out = pl.pallas_call(
        _out_body,
        grid=(1,),
        in_specs=[_full_spec((N, HIDDEN)), _full_spec((N, _AF_PAD)),
                  _full_spec((OUT, HIDDEN)), _full_spec((OUT, _AF_PAD))],
        out_specs=_full_spec((OUT, N)),
        out_shape=jax.ShapeDtypeStruct((OUT, N), jnp.float32),
        compiler_params=params,
    )(s_h, s_a, V_h, va64)
    return out
